# baseline (speedup 1.0000x reference)
import functools

import jax
import jax.numpy as jnp
from jax import lax
from jax.experimental import pallas as pl
from jax.experimental.pallas import tpu as pltpu
from jax.experimental.pallas import tpu_sc as plsc

B = 64
P = 8192
D = 128
K = 64
NSLAB = P // 128


def _sim_body(lf_ref, proto_ref, nrt_ref, sim_ref):
    p = proto_ref[0]
    pn = p / (jnp.sqrt(jnp.sum(p * p)) + 1e-8)
    pnb = pn.astype(jnp.bfloat16).astype(jnp.float32)
    x3 = lf_ref[0].reshape(NSLAB, 128, D)
    nrt = nrt_ref[0]
    for i in range(NSLAB):
        col = nrt[:, i:i + 1]
        ln = x3[i] / (col + 1e-8)
        lnb = ln.astype(jnp.bfloat16).astype(jnp.float32)
        sim_ref[0, :, i:i + 1] = jnp.sum(lnb * pnb, axis=1, keepdims=True)


def _sim(local_features, prototypes):
    nrm = jnp.linalg.norm(local_features, axis=-1)
    nrt = nrm.reshape(B, NSLAB, 128).transpose(0, 2, 1)
    out = pl.pallas_call(
        _sim_body,
        grid=(B,),
        in_specs=[
            pl.BlockSpec((1, P, D), lambda b: (b, 0, 0)),
            pl.BlockSpec((1, 1, D), lambda b: (b, 0, 0)),
            pl.BlockSpec((1, 128, NSLAB), lambda b: (b, 0, 0)),
        ],
        out_specs=pl.BlockSpec((1, 128, NSLAB), lambda b: (b, 0, 0)),
        out_shape=jax.ShapeDtypeStruct((B, 128, NSLAB), jnp.float32),
    )(local_features, prototypes, nrt)
    return out.reshape(B, P)


def _topk_body(sim_ref, ti_ref, bi_ref, ex_ref):
    iq = lax.broadcasted_iota(jnp.int32, (B, P), 1)
    iop = ((iq & (NSLAB - 1)) << 7) | (iq >> 6)
    kio = lax.broadcasted_iota(jnp.int32, (B, K), 1)
    inf = jnp.float32(jnp.inf)
    s0 = sim_ref[...]
    ex_ref[...] = jnp.zeros((B, P), jnp.int8)
    vt0 = jnp.max(s0, axis=1, keepdims=True)
    vb0 = jnp.min(s0, axis=1, keepdims=True)

    def step(k, carry):
        ti, bi, vt, vb = carry
        st = sim_ref[...]
        ex = ex_ref[...] != 0
        it = jnp.min(jnp.where((st == vt) & ~ex, iop, P), axis=1, keepdims=True)
        ib = jnp.min(jnp.where((st == vb) & ~ex, iop, P), axis=1, keepdims=True)
        st2 = sim_ref[...]
        hit = (iop == it) | (iop == ib)
        ex2 = ex_ref[...] != 0
        exn = ex2 | hit
        ex_ref[...] = exn.astype(jnp.int8)
        stm = jnp.where(exn, -inf, st2)
        sbm = jnp.where(exn, inf, st2)
        vt2 = jnp.max(stm, axis=1, keepdims=True)
        vb2 = jnp.min(sbm, axis=1, keepdims=True)
        sel = kio == k
        ti = jnp.where(sel, it, ti)
        bi = jnp.where(sel, ib, bi)
        return ti, bi, vt2, vb2

    zero = jnp.zeros((B, K), jnp.int32)
    ti, bi, _, _ = lax.fori_loop(0, K, step, (zero, zero, vt0, vb0))
    ti_ref[...] = ti
    bi_ref[...] = bi


def _topk(sim):
    return pl.pallas_call(
        _topk_body,
        out_shape=[
            jax.ShapeDtypeStruct((B, K), jnp.int32),
            jax.ShapeDtypeStruct((B, K), jnp.int32),
        ],
        scratch_shapes=[
            pltpu.VMEM((B, P), jnp.int8),
        ],
    )(sim)


_NROWS = 2 * B * K


@functools.cache
def _make_sc_gather():
    info = plsc.get_sparse_core_info()
    nw = info.num_cores * info.num_subcores
    bpw = _NROWS // nw
    mesh = plsc.VectorSubcoreMesh(core_axis_name="c", subcore_axis_name="s")

    @functools.partial(
        pl.kernel,
        mesh=mesh,
        out_type=jax.ShapeDtypeStruct((_NROWS, D), jnp.float32),
        scratch_types=[
            pltpu.VMEM((bpw,), jnp.int32),
            pltpu.VMEM((bpw, D), jnp.float32),
            pltpu.SemaphoreType.DMA,
        ],
    )
    def gather(table_hbm, idx_hbm, out_hbm, idx_v, rows_v, sem):
        wid = lax.axis_index("s") * info.num_cores + lax.axis_index("c")
        base = wid * bpw
        pltpu.sync_copy(idx_hbm.at[pl.ds(base, bpw)], idx_v)
        pltpu.async_copy(table_hbm.at[idx_v], rows_v, sem).wait()
        pltpu.sync_copy(rows_v, out_hbm.at[pl.ds(base, bpw)])

    return gather


@jax.jit
def kernel(local_features, prototypes):
    sim = _sim(local_features, prototypes)
    ti, bi = _topk(sim)
    offs = (jnp.arange(B, dtype=jnp.int32) * P)[:, None]
    flat_idx = jnp.concatenate([ti + offs, bi + offs], axis=0).reshape(-1)
    table = local_features.reshape(B * P, D)
    rows = _make_sc_gather()(table, flat_idx).reshape(2, B, K, D)
    return rows[0], rows[1], ti, bi


# speedup vs baseline: 1.1233x; 1.1233x over previous
import functools

import jax
import jax.numpy as jnp
from jax import lax
from jax.experimental import pallas as pl
from jax.experimental.pallas import tpu as pltpu
from jax.experimental.pallas import tpu_sc as plsc

B = 64
P = 8192
D = 128
K = 64
NSLAB = P // 128


def _sim_body(lf_ref, proto_ref, nrt_ref, sim_ref):
    p = proto_ref[0]
    pn = p / (jnp.sqrt(jnp.sum(p * p)) + 1e-8)
    pnb = pn.astype(jnp.bfloat16).astype(jnp.float32)
    x3 = lf_ref[0].reshape(NSLAB, 128, D)
    nrt = nrt_ref[0]
    for i in range(NSLAB):
        col = nrt[:, i:i + 1]
        ln = x3[i] / (col + 1e-8)
        lnb = ln.astype(jnp.bfloat16).astype(jnp.float32)
        sim_ref[0, :, i:i + 1] = jnp.sum(lnb * pnb, axis=1, keepdims=True)


def _sim(local_features, prototypes):
    nrm = jnp.linalg.norm(local_features, axis=-1)
    nrt = nrm.reshape(B, NSLAB, 128).transpose(0, 2, 1)
    out = pl.pallas_call(
        _sim_body,
        grid=(B,),
        in_specs=[
            pl.BlockSpec((1, P, D), lambda b: (b, 0, 0)),
            pl.BlockSpec((1, 1, D), lambda b: (b, 0, 0)),
            pl.BlockSpec((1, 128, NSLAB), lambda b: (b, 0, 0)),
        ],
        out_specs=pl.BlockSpec((1, 128, NSLAB), lambda b: (b, 0, 0)),
        out_shape=jax.ShapeDtypeStruct((B, 128, NSLAB), jnp.float32),
    )(local_features, prototypes, nrt)
    return out.reshape(B, P)


def _topk_body(sim_ref, ti_ref, bi_ref, st_ref, sb_ref):
    iq = lax.broadcasted_iota(jnp.int32, (B, P), 1)
    iop = ((iq & (NSLAB - 1)) << 7) | (iq >> 6)
    kio = lax.broadcasted_iota(jnp.int32, (B, K), 1)
    inf = jnp.float32(jnp.inf)
    s0 = sim_ref[...]
    st_ref[...] = s0
    sb_ref[...] = s0
    vt0 = jnp.max(s0, axis=1, keepdims=True)
    vb0 = jnp.min(s0, axis=1, keepdims=True)

    def step(k, carry):
        ti, bi, vt, vb = carry
        st = st_ref[...]
        sb = sb_ref[...]
        it = jnp.min(jnp.where(st == vt, iop, P), axis=1, keepdims=True)
        ib = jnp.min(jnp.where(sb == vb, iop, P), axis=1, keepdims=True)
        st2 = jnp.where(iop == it, -inf, st)
        sb2 = jnp.where(iop == ib, inf, sb)
        st_ref[...] = st2
        sb_ref[...] = sb2
        vt2 = jnp.max(st2, axis=1, keepdims=True)
        vb2 = jnp.min(sb2, axis=1, keepdims=True)
        sel = kio == k
        ti = jnp.where(sel, it, ti)
        bi = jnp.where(sel, ib, bi)
        return ti, bi, vt2, vb2

    zero = jnp.zeros((B, K), jnp.int32)
    ti, bi, _, _ = lax.fori_loop(0, K, step, (zero, zero, vt0, vb0))
    ti_ref[...] = ti
    bi_ref[...] = bi


def _topk(sim):
    return pl.pallas_call(
        _topk_body,
        out_shape=[
            jax.ShapeDtypeStruct((B, K), jnp.int32),
            jax.ShapeDtypeStruct((B, K), jnp.int32),
        ],
        scratch_shapes=[
            pltpu.VMEM((B, P), jnp.float32),
            pltpu.VMEM((B, P), jnp.float32),
        ],
    )(sim)


_NROWS = 2 * B * K


@functools.cache
def _make_sc_gather():
    info = plsc.get_sparse_core_info()
    nw = info.num_cores * info.num_subcores
    bpw = _NROWS // nw
    mesh = plsc.VectorSubcoreMesh(core_axis_name="c", subcore_axis_name="s")

    @functools.partial(
        pl.kernel,
        mesh=mesh,
        out_type=jax.ShapeDtypeStruct((_NROWS, D), jnp.float32),
        scratch_types=[
            pltpu.VMEM((bpw,), jnp.int32),
            pltpu.VMEM((bpw, D), jnp.float32),
            pltpu.SemaphoreType.DMA,
        ],
    )
    def gather(table_hbm, idx_hbm, out_hbm, idx_v, rows_v, sem):
        wid = lax.axis_index("s") * info.num_cores + lax.axis_index("c")
        base = wid * bpw
        pltpu.sync_copy(idx_hbm.at[pl.ds(base, bpw)], idx_v)
        pltpu.async_copy(table_hbm.at[idx_v], rows_v, sem).wait()
        pltpu.sync_copy(rows_v, out_hbm.at[pl.ds(base, bpw)])

    return gather


@jax.jit
def kernel(local_features, prototypes):
    sim = _sim(local_features, prototypes)
    ti, bi = _topk(sim)
    offs = (jnp.arange(B, dtype=jnp.int32) * P)[:, None]
    flat_idx = jnp.concatenate([ti + offs, bi + offs], axis=0).reshape(-1)
    table = local_features.reshape(B * P, D)
    rows = _make_sc_gather()(table, flat_idx).reshape(2, B, K, D)
    return rows[0], rows[1], ti, bi


# speedup vs baseline: 1.1425x; 1.0171x over previous
import functools

import jax
import jax.numpy as jnp
from jax import lax
from jax.experimental import pallas as pl
from jax.experimental.pallas import tpu as pltpu
from jax.experimental.pallas import tpu_sc as plsc

B = 64
P = 8192
D = 128
K = 64
NSLAB = P // 128


def _sim_body(lf_ref, proto_ref, nrt_ref, sim_ref):
    p = proto_ref[0]
    pn = p / (jnp.sqrt(jnp.sum(p * p)) + 1e-8)
    pnb = pn.astype(jnp.bfloat16).astype(jnp.float32)
    x3 = lf_ref[0].reshape(NSLAB, 128, D)
    nrt = nrt_ref[0]
    for i in range(NSLAB):
        col = nrt[:, i:i + 1]
        ln = x3[i] / (col + 1e-8)
        lnb = ln.astype(jnp.bfloat16).astype(jnp.float32)
        sim_ref[0, :, i:i + 1] = jnp.sum(lnb * pnb, axis=1, keepdims=True)


def _sim(local_features, prototypes):
    nrm = jnp.linalg.norm(local_features, axis=-1)
    nrt = nrm.reshape(B, NSLAB, 128).transpose(0, 2, 1)
    out = pl.pallas_call(
        _sim_body,
        grid=(B,),
        in_specs=[
            pl.BlockSpec((1, P, D), lambda b: (b, 0, 0)),
            pl.BlockSpec((1, 1, D), lambda b: (b, 0, 0)),
            pl.BlockSpec((1, 128, NSLAB), lambda b: (b, 0, 0)),
        ],
        out_specs=pl.BlockSpec((1, 128, NSLAB), lambda b: (b, 0, 0)),
        out_shape=jax.ShapeDtypeStruct((B, 128, NSLAB), jnp.float32),
    )(local_features, prototypes, nrt)
    return out.transpose(0, 2, 1).reshape(B, P)


def _topk_body(sim_ref, ti_ref, bi_ref, st_ref, sb_ref):
    iota = lax.broadcasted_iota(jnp.int32, (B, P), 1)
    kio = lax.broadcasted_iota(jnp.int32, (B, K), 1)
    inf = jnp.float32(jnp.inf)
    st_ref[...] = sim_ref[...]
    sb_ref[...] = sim_ref[...]

    def step(k, carry):
        ti, bi = carry
        st = st_ref[...]
        sb = sb_ref[...]
        vt = jnp.max(st, axis=1, keepdims=True)
        it = jnp.min(jnp.where(st == vt, iota, P), axis=1, keepdims=True)
        vb = jnp.min(sb, axis=1, keepdims=True)
        ib = jnp.min(jnp.where(sb == vb, iota, P), axis=1, keepdims=True)
        st_ref[...] = jnp.where(iota == it, -inf, st)
        sb_ref[...] = jnp.where(iota == ib, inf, sb)
        sel = kio == k
        ti = jnp.where(sel, it, ti)
        bi = jnp.where(sel, ib, bi)
        return ti, bi

    zero = jnp.zeros((B, K), jnp.int32)
    ti, bi = lax.fori_loop(0, K, step, (zero, zero))
    ti_ref[...] = ti
    bi_ref[...] = bi


def _topk(sim):
    return pl.pallas_call(
        _topk_body,
        out_shape=[
            jax.ShapeDtypeStruct((B, K), jnp.int32),
            jax.ShapeDtypeStruct((B, K), jnp.int32),
        ],
        scratch_shapes=[
            pltpu.VMEM((B, P), jnp.float32),
            pltpu.VMEM((B, P), jnp.float32),
        ],
    )(sim)


_NROWS = 2 * B * K


@functools.cache
def _make_sc_gather():
    info = plsc.get_sparse_core_info()
    nw = info.num_cores * info.num_subcores
    bpw = _NROWS // nw
    mesh = plsc.VectorSubcoreMesh(core_axis_name="c", subcore_axis_name="s")

    @functools.partial(
        pl.kernel,
        mesh=mesh,
        out_type=jax.ShapeDtypeStruct((_NROWS, D), jnp.float32),
        scratch_types=[
            pltpu.VMEM((bpw,), jnp.int32),
            pltpu.VMEM((bpw, D), jnp.float32),
            pltpu.SemaphoreType.DMA,
        ],
    )
    def gather(table_hbm, idx_hbm, out_hbm, idx_v, rows_v, sem):
        wid = lax.axis_index("s") * info.num_cores + lax.axis_index("c")
        base = wid * bpw
        pltpu.sync_copy(idx_hbm.at[pl.ds(base, bpw)], idx_v)
        pltpu.async_copy(table_hbm.at[idx_v], rows_v, sem).wait()
        pltpu.sync_copy(rows_v, out_hbm.at[pl.ds(base, bpw)])

    return gather


@jax.jit
def kernel(local_features, prototypes):
    sim = _sim(local_features, prototypes)
    ti, bi = _topk(sim)
    offs = (jnp.arange(B, dtype=jnp.int32) * P)[:, None]
    flat_idx = jnp.concatenate([ti + offs, bi + offs], axis=0).reshape(-1)
    table = local_features.reshape(B * P, D)
    rows = _make_sc_gather()(table, flat_idx).reshape(2, B, K, D)
    return rows[0], rows[1], ti, bi


# speedup vs baseline: 1.2373x; 1.0830x over previous
import functools

import jax
import jax.numpy as jnp
from jax import lax
from jax.experimental import pallas as pl
from jax.experimental.pallas import tpu as pltpu
from jax.experimental.pallas import tpu_sc as plsc

B = 64
P = 8192
D = 128
K = 64


def _sim_body(lf_ref, proto_ref, nrm_ref, sim_ref):
    x = lf_ref[0]
    p = proto_ref[0]
    pn = p / (jnp.sqrt(jnp.sum(p * p)) + 1e-8)
    ln = x / (nrm_ref[0] + 1e-8)
    lnb = ln.astype(jnp.bfloat16).astype(jnp.float32)
    pnb = pn.astype(jnp.bfloat16).astype(jnp.float32)
    sim_ref[0, 0] = jnp.sum(lnb * pnb, axis=1)


def _sim(local_features, prototypes):
    nrm = jnp.linalg.norm(local_features, axis=-1, keepdims=True)
    out = pl.pallas_call(
        _sim_body,
        grid=(B,),
        in_specs=[
            pl.BlockSpec((1, P, D), lambda b: (b, 0, 0)),
            pl.BlockSpec((1, 1, D), lambda b: (b, 0, 0)),
            pl.BlockSpec((1, P, 1), lambda b: (b, 0, 0)),
        ],
        out_specs=pl.BlockSpec((1, 1, P), lambda b: (b, 0, 0)),
        out_shape=jax.ShapeDtypeStruct((B, 1, P), jnp.float32),
    )(local_features, prototypes, nrm)
    return out.reshape(B, P)


def _topk_body(sim_ref, ti_ref, bi_ref, st_ref, sb_ref):
    iota = lax.broadcasted_iota(jnp.int32, (B, P), 1)
    kio = lax.broadcasted_iota(jnp.int32, (B, K), 1)
    inf = jnp.float32(jnp.inf)
    st_ref[...] = sim_ref[...]
    sb_ref[...] = sim_ref[...]

    def step(k, carry):
        ti, bi = carry
        st = st_ref[...]
        sb = sb_ref[...]
        vt = jnp.max(st, axis=1, keepdims=True)
        it = jnp.min(jnp.where(st == vt, iota, P), axis=1, keepdims=True)
        vb = jnp.min(sb, axis=1, keepdims=True)
        ib = jnp.min(jnp.where(sb == vb, iota, P), axis=1, keepdims=True)
        st_ref[...] = jnp.where(iota == it, -inf, st)
        sb_ref[...] = jnp.where(iota == ib, inf, sb)
        sel = kio == k
        ti = jnp.where(sel, it, ti)
        bi = jnp.where(sel, ib, bi)
        return ti, bi

    zero = jnp.zeros((B, K), jnp.int32)
    ti, bi = lax.fori_loop(0, K, step, (zero, zero))
    ti_ref[...] = ti
    bi_ref[...] = bi


def _topk(sim):
    return pl.pallas_call(
        _topk_body,
        out_shape=[
            jax.ShapeDtypeStruct((B, K), jnp.int32),
            jax.ShapeDtypeStruct((B, K), jnp.int32),
        ],
        scratch_shapes=[
            pltpu.VMEM((B, P), jnp.float32),
            pltpu.VMEM((B, P), jnp.float32),
        ],
    )(sim)


_NROWS = 2 * B * K


@functools.cache
def _make_sc_gather():
    info = plsc.get_sparse_core_info()
    nw = info.num_cores * info.num_subcores
    bpw = _NROWS // nw
    mesh = plsc.VectorSubcoreMesh(core_axis_name="c", subcore_axis_name="s")

    @functools.partial(
        pl.kernel,
        mesh=mesh,
        out_type=jax.ShapeDtypeStruct((_NROWS, D), jnp.float32),
        scratch_types=[
            pltpu.VMEM((bpw,), jnp.int32),
            pltpu.VMEM((bpw, D), jnp.float32),
            pltpu.SemaphoreType.DMA,
        ],
    )
    def gather(table_hbm, idx_hbm, out_hbm, idx_v, rows_v, sem):
        wid = lax.axis_index("s") * info.num_cores + lax.axis_index("c")
        base = wid * bpw
        pltpu.sync_copy(idx_hbm.at[pl.ds(base, bpw)], idx_v)
        pltpu.async_copy(table_hbm.at[idx_v], rows_v, sem).wait()
        pltpu.sync_copy(rows_v, out_hbm.at[pl.ds(base, bpw)])

    return gather


@jax.jit
def kernel(local_features, prototypes):
    sim = _sim(local_features, prototypes)
    ti, bi = _topk(sim)
    offs = (jnp.arange(B, dtype=jnp.int32) * P)[:, None]
    flat_idx = jnp.concatenate([ti + offs, bi + offs], axis=0).reshape(-1)
    table = local_features.reshape(B * P, D)
    rows = _make_sc_gather()(table, flat_idx).reshape(2, B, K, D)
    return rows[0], rows[1], ti, bi


# speedup vs baseline: 1.3108x; 1.0594x over previous
import functools

import jax
import jax.numpy as jnp
from jax import lax
from jax.experimental import pallas as pl
from jax.experimental.pallas import tpu as pltpu
from jax.experimental.pallas import tpu_sc as plsc

B = 64
P = 8192
D = 128
K = 64


def _sim_body(lf_ref, proto_ref, nrm_ref, sim_ref):
    x = lf_ref[0]
    p = proto_ref[0]
    pn = p / (jnp.sqrt(jnp.sum(p * p)) + 1e-8)
    nrc = jnp.transpose(nrm_ref[0], (1, 0))
    ln = x / (nrc + 1e-8)
    lnb = ln.astype(jnp.bfloat16).astype(jnp.float32)
    pnb = pn.astype(jnp.bfloat16).astype(jnp.float32)
    sim_ref[0, 0] = jnp.sum(lnb * pnb, axis=1)


def _sim(local_features, prototypes):
    nrm = jnp.linalg.norm(local_features, axis=-1)[:, None, :]
    out = pl.pallas_call(
        _sim_body,
        grid=(B,),
        in_specs=[
            pl.BlockSpec((1, P, D), lambda b: (b, 0, 0)),
            pl.BlockSpec((1, 1, D), lambda b: (b, 0, 0)),
            pl.BlockSpec((1, 1, P), lambda b: (b, 0, 0)),
        ],
        out_specs=pl.BlockSpec((1, 1, P), lambda b: (b, 0, 0)),
        out_shape=jax.ShapeDtypeStruct((B, 1, P), jnp.float32),
    )(local_features, prototypes, nrm)
    return out.reshape(B, P)


def _topk_body(sim_ref, ti_ref, bi_ref, st_ref, sb_ref):
    iota = lax.broadcasted_iota(jnp.int32, (B, P), 1)
    kio = lax.broadcasted_iota(jnp.int32, (B, K), 1)
    inf = jnp.float32(jnp.inf)
    st_ref[...] = sim_ref[...]
    sb_ref[...] = sim_ref[...]

    def step(k, carry):
        ti, bi = carry
        st = st_ref[...]
        sb = sb_ref[...]
        vt = jnp.max(st, axis=1, keepdims=True)
        it = jnp.min(jnp.where(st == vt, iota, P), axis=1, keepdims=True)
        vb = jnp.min(sb, axis=1, keepdims=True)
        ib = jnp.min(jnp.where(sb == vb, iota, P), axis=1, keepdims=True)
        st_ref[...] = jnp.where(iota == it, -inf, st)
        sb_ref[...] = jnp.where(iota == ib, inf, sb)
        sel = kio == k
        ti = jnp.where(sel, it, ti)
        bi = jnp.where(sel, ib, bi)
        return ti, bi

    zero = jnp.zeros((B, K), jnp.int32)
    ti, bi = lax.fori_loop(0, K, step, (zero, zero))
    ti_ref[...] = ti
    bi_ref[...] = bi


def _topk(sim):
    return pl.pallas_call(
        _topk_body,
        out_shape=[
            jax.ShapeDtypeStruct((B, K), jnp.int32),
            jax.ShapeDtypeStruct((B, K), jnp.int32),
        ],
        scratch_shapes=[
            pltpu.VMEM((B, P), jnp.float32),
            pltpu.VMEM((B, P), jnp.float32),
        ],
    )(sim)


_NROWS = 2 * B * K


@functools.cache
def _make_sc_gather():
    info = plsc.get_sparse_core_info()
    nw = info.num_cores * info.num_subcores
    bpw = _NROWS // nw
    mesh = plsc.VectorSubcoreMesh(core_axis_name="c", subcore_axis_name="s")

    @functools.partial(
        pl.kernel,
        mesh=mesh,
        out_type=jax.ShapeDtypeStruct((_NROWS, D), jnp.float32),
        scratch_types=[
            pltpu.VMEM((bpw,), jnp.int32),
            pltpu.VMEM((bpw, D), jnp.float32),
            pltpu.SemaphoreType.DMA,
        ],
    )
    def gather(table_hbm, idx_hbm, out_hbm, idx_v, rows_v, sem):
        wid = lax.axis_index("s") * info.num_cores + lax.axis_index("c")
        base = wid * bpw
        pltpu.sync_copy(idx_hbm.at[pl.ds(base, bpw)], idx_v)
        pltpu.async_copy(table_hbm.at[idx_v], rows_v, sem).wait()
        pltpu.sync_copy(rows_v, out_hbm.at[pl.ds(base, bpw)])

    return gather


@jax.jit
def kernel(local_features, prototypes):
    sim = _sim(local_features, prototypes)
    ti, bi = _topk(sim)
    offs = (jnp.arange(B, dtype=jnp.int32) * P)[:, None]
    flat_idx = jnp.concatenate([ti + offs, bi + offs], axis=0).reshape(-1)
    table = local_features.reshape(B * P, D)
    rows = _make_sc_gather()(table, flat_idx).reshape(2, B, K, D)
    return rows[0], rows[1], ti, bi
